# baseline (speedup 1.0000x reference)
import functools

import jax
import jax.numpy as jnp
from jax import lax
from jax.experimental import pallas as pl
from jax.experimental.pallas import tpu as pltpu
from jax.experimental.pallas import tpu_sc as plsc

NW = 32
BC = 104
NB = 13


def _mm_body(nj, x_ref, w_ref, b_ref, o_ref):
    x = x_ref[...]
    for j in range(nj):
        y = jnp.dot(x, w_ref[j], preferred_element_type=jnp.float32)
        if j == 0:
            y = y + b_ref[...]
        o_ref[j] = y


def _matmul_tables(x, wgrp, brow):
    npad, cin = x.shape
    nj = wgrp.shape[0]
    bn = npad // NB
    return pl.pallas_call(
        functools.partial(_mm_body, nj),
        grid=(NB,),
        in_specs=[
            pl.BlockSpec((bn, cin), lambda i: (i, 0)),
            pl.BlockSpec((nj, cin, 128), lambda i: (0, 0, 0)),
            pl.BlockSpec((1, 128), lambda i: (0, 0)),
        ],
        out_specs=pl.BlockSpec((nj, bn, 128), lambda i: (0, i, 0)),
        out_shape=jax.ShapeDtypeStruct((nj, npad, 128), jnp.float32),
        compiler_params=pltpu.CompilerParams(
            dimension_semantics=("parallel",)
        ),
    )(x, wgrp, brow)


def _gather_sum(idxc, table, oc, act, npad):
    nch = idxc.shape[1]
    kk = idxc.shape[2]
    mesh = plsc.VectorSubcoreMesh(core_axis_name="c", subcore_axis_name="s")

    @functools.partial(
        pl.kernel,
        out_type=jax.ShapeDtypeStruct((npad, oc), jnp.float32),
        mesh=mesh,
        scratch_types=(
            [pltpu.VMEM((kk, BC), jnp.int32) for _ in range(2)]
            + [pltpu.VMEM((BC, oc), jnp.float32) for _ in range(2 * kk)]
            + [pltpu.VMEM((BC, oc), jnp.float32) for _ in range(2)]
            + [pltpu.SemaphoreType.DMA for _ in range(4)]
        ),
        compiler_params=pltpu.CompilerParams(use_tc_tiling_on_sc=False),
    )
    def run(idx_hbm, table_hbm, out_hbm, *rest):
        idx_vs = rest[0:2]
        bufs = [rest[2:2 + kk], rest[2 + kk:2 + 2 * kk]]
        out_vs = rest[2 + 2 * kk:4 + 2 * kk]
        gsems = rest[4 + 2 * kk:6 + 2 * kk]
        osems = rest[6 + 2 * kk:8 + 2 * kk]
        wid = lax.axis_index("s") * 2 + lax.axis_index("c")
        base0 = wid * (nch * BC)
        nhalf = nch // 2

        def fire(c, s):
            pltpu.sync_copy(idx_hbm.at[wid, c], idx_vs[s])
            return [
                pltpu.async_copy(table_hbm.at[idx_vs[s].at[j]], bufs[s][j],
                                 gsems[s])
                for j in range(kk)
            ]

        def wait_gather(s):
            for j in range(kk):
                pltpu.make_async_copy(
                    table_hbm.at[idx_vs[s].at[j]], bufs[s][j], gsems[s]
                ).wait()

        def owrite(c, s):
            return pltpu.async_copy(
                out_vs[s], out_hbm.at[pl.ds(base0 + c * BC, BC)], osems[s])

        def accumulate(s):
            def row(r, carry):
                for c2 in range(oc // 16):
                    sl = pl.ds(c2 * 16, 16)
                    v = bufs[s][0][r, sl]
                    for j in range(1, kk):
                        v = v + bufs[s][j][r, sl]
                    if act:
                        v = 1.0 / (1.0 + jnp.exp(-v))
                    out_vs[s][r, sl] = v
                return carry

            lax.fori_loop(0, BC, row, 0)

        fire(0, 0)
        fire(1, 1)
        wait_gather(0)
        accumulate(0)
        owrite(0, 0)
        if nhalf > 1:
            fire(2, 0)
        wait_gather(1)
        accumulate(1)
        owrite(1, 1)
        if nhalf > 1:
            fire(3, 1)

        def pair(m, carry):
            c0 = 2 * m
            for s in range(2):
                c = c0 + s
                wait_gather(s)
                pltpu.make_async_copy(
                    out_vs[s],
                    out_hbm.at[pl.ds(base0 + (c - 2) * BC, BC)],
                    osems[s],
                ).wait()
                accumulate(s)
                owrite(c, s)

                @pl.when(m < nhalf - 1)
                def _():
                    fire(c + 2, s)

            return carry

        lax.fori_loop(1, nhalf, pair, 0)
        for s in range(2):
            pltpu.make_async_copy(
                out_vs[s],
                out_hbm.at[pl.ds(base0 + (nch - 2 + s) * BC, BC)],
                osems[s],
            ).wait()

    return run(idxc, table)


def kernel(features, knn_indices, W0, b0, W1, b1, W2, b2):
    n, _ = features.shape
    k = knn_indices.shape[1]
    nch = -(-n // (NW * BC))
    npad = NW * BC * nch

    x = jnp.pad(features, ((0, npad - n), (0, 0)))
    idx = jnp.pad(knn_indices, ((0, npad - n), (0, 0)))

    w2p = jnp.pad(W2, ((0, 16 - W2.shape[0]), (0, 0)))
    b2p = jnp.pad(b2, ((0, 0), (0, 16 - b2.shape[1])))

    kr = jnp.arange(k, dtype=jnp.int32)
    h = x
    for wgt, bias, act in ((W0, b0, True), (W1, b1, True), (w2p, b2p, False)):
        oc = wgt.shape[0]
        cin = h.shape[1]
        g = 128 // oc
        wk = wgt.reshape(oc, k, cin).transpose(1, 2, 0)
        wgrp = wk.reshape(k // g, g, cin, oc).transpose(0, 2, 1, 3)
        wgrp = wgrp.reshape(k // g, cin, 128)
        brow = jnp.pad(bias, ((0, 0), (0, 128 - oc)))
        idxa = idx * g + ((kr // g) * npad * g + kr % g)[None, :]
        idxc = idxa.reshape(NW, nch, BC, k).transpose(0, 1, 3, 2)
        y = _matmul_tables(h, wgrp, brow)
        table = y.reshape(npad * k, oc)
        h = _gather_sum(idxc, table, oc, act, npad)

    return h[:n, :3]


# speedup vs baseline: 1.6609x; 1.6609x over previous
import functools

import jax
import jax.numpy as jnp
from jax import lax
from jax.experimental import pallas as pl
from jax.experimental.pallas import tpu as pltpu
from jax.experimental.pallas import tpu_sc as plsc

NW = 32
BC = 112
NB = 14


def _mm_body(nj, x_ref, w_ref, b_ref, o_ref):
    x = x_ref[...]
    for j in range(nj):
        y = jnp.dot(x, w_ref[j], preferred_element_type=jnp.float32)
        if j == 0:
            y = y + b_ref[...]
        o_ref[j] = y


def _matmul_tables(x, wbd, brow, rows_out):
    cx = x.shape[1]
    nj, _, wout = wbd.shape
    bn = rows_out // NB
    return pl.pallas_call(
        functools.partial(_mm_body, nj),
        grid=(NB,),
        in_specs=[
            pl.BlockSpec((bn, cx), lambda i: (i, 0)),
            pl.BlockSpec((nj, cx, wout), lambda i: (0, 0, 0)),
            pl.BlockSpec((1, wout), lambda i: (0, 0)),
        ],
        out_specs=pl.BlockSpec((nj, bn, wout), lambda i: (0, i, 0)),
        out_shape=jax.ShapeDtypeStruct((nj, rows_out, wout), jnp.float32),
        compiler_params=pltpu.CompilerParams(
            dimension_semantics=("parallel",)
        ),
    )(x, wbd, brow)


def _gather_sum(idxc, table, oc, act, npad, qout):
    nch = idxc.shape[1]
    kk = idxc.shape[2]
    ow = qout * oc
    mesh = plsc.VectorSubcoreMesh(core_axis_name="c", subcore_axis_name="s")

    @functools.partial(
        pl.kernel,
        out_type=jax.ShapeDtypeStruct((npad // qout, ow), jnp.float32),
        mesh=mesh,
        scratch_types=(
            [pltpu.VMEM((kk, BC), jnp.int32)]
            + [pltpu.VMEM((BC, oc), jnp.float32) for _ in range(kk)]
            + [pltpu.VMEM((BC // qout, ow), jnp.float32),
               pltpu.SemaphoreType.DMA]
        ),
        compiler_params=pltpu.CompilerParams(use_tc_tiling_on_sc=False),
    )
    def run(idx_hbm, table_hbm, out_hbm, idx_v, *rest):
        bufs = rest[:kk]
        out_v = rest[kk]
        sem = rest[kk + 1]
        wid = lax.axis_index("s") * 2 + lax.axis_index("c")
        base0 = wid * (nch * BC)
        for c in range(nch):
            base = base0 + c * BC
            pltpu.sync_copy(idx_hbm.at[wid, c], idx_v)
            cps = [
                pltpu.async_copy(table_hbm.at[idx_v.at[j]], bufs[j], sem)
                for j in range(kk)
            ]
            for cp in cps:
                cp.wait()

            def row(rr, carry):
                for c2 in range(ow // 16):
                    node = rr * qout + (c2 * 16) // oc
                    col = (c2 * 16) % oc
                    s = bufs[0][node, pl.ds(col, 16)]
                    for j in range(1, kk):
                        s = s + bufs[j][node, pl.ds(col, 16)]
                    if act:
                        s = 1.0 / (1.0 + jnp.exp(-s))
                    out_v[rr, pl.ds(c2 * 16, 16)] = s
                return carry

            lax.fori_loop(0, BC // qout, row, 0)
            pltpu.sync_copy(
                out_v, out_hbm.at[pl.ds(base // qout, BC // qout)])

    return run(idxc, table)


def kernel(features, knn_indices, W0, b0, W1, b1, W2, b2):
    n, _ = features.shape
    k = knn_indices.shape[1]
    nch = -(-n // (NW * BC))
    npad = NW * BC * nch

    idx = jnp.pad(knn_indices, ((0, npad - n), (0, 0)))

    w2p = jnp.pad(W2, ((0, 16 - W2.shape[0]), (0, 0)))
    b2p = jnp.pad(b2, ((0, 0), (0, 16 - b2.shape[1])))

    kr = jnp.arange(k, dtype=jnp.int32)
    eye = {q: jnp.eye(q, dtype=jnp.float32) for q in (1, 2, 4)}
    h = features
    layers = (
        (W0, b0, True, 1, 2),
        (W1, b1, True, 2, 4),
        (w2p, b2p, False, 4, 1),
    )
    for wgt, bias, act, qin, qout in layers:
        oc = wgt.shape[0]
        cin = wgt.shape[1] // k
        g = 128 // oc
        wk = wgt.reshape(oc, k, cin).transpose(1, 2, 0)
        wgrp = wk.reshape(k // g, g, cin, oc).transpose(0, 2, 1, 3)
        wgrp = wgrp.reshape(k // g, cin, 128)
        wbd = jax.vmap(lambda w: jnp.kron(eye[qin], w))(wgrp)
        brow = jnp.tile(jnp.pad(bias, ((0, 0), (0, 128 - oc))), (1, qin))
        idxa = idx * g + ((kr // g) * npad * g + kr % g)[None, :]
        idxc = idxa.reshape(NW, nch, BC, k).transpose(0, 1, 3, 2)
        y = _matmul_tables(h, wbd, brow, npad // qin)
        table = y.reshape(npad * k, oc)
        h = _gather_sum(idxc, table, oc, act, npad, qout)

    return h[:n, :3]


# speedup vs baseline: 1.6689x; 1.0049x over previous
import functools

import jax
import jax.numpy as jnp
from jax import lax
from jax.experimental import pallas as pl
from jax.experimental.pallas import tpu as pltpu
from jax.experimental.pallas import tpu_sc as plsc

NW = 32
BC = 112
NB = 14


def _mm_body(nj, x_ref, w_ref, b_ref, o_ref):
    x = x_ref[...]
    for j in range(nj):
        y = jnp.dot(x, w_ref[j], preferred_element_type=jnp.float32)
        if j == 0:
            y = y + b_ref[...]
        o_ref[j] = y


def _matmul_tables(x, wbd, brow, rows_out):
    cx = x.shape[1]
    nj, _, wout = wbd.shape
    bn = rows_out // NB
    return pl.pallas_call(
        functools.partial(_mm_body, nj),
        grid=(NB,),
        in_specs=[
            pl.BlockSpec((bn, cx), lambda i: (i, 0)),
            pl.BlockSpec((nj, cx, wout), lambda i: (0, 0, 0)),
            pl.BlockSpec((1, wout), lambda i: (0, 0)),
        ],
        out_specs=pl.BlockSpec((nj, bn, wout), lambda i: (0, i, 0)),
        out_shape=jax.ShapeDtypeStruct((nj, rows_out, wout), jnp.float32),
        compiler_params=pltpu.CompilerParams(
            dimension_semantics=("parallel",)
        ),
    )(x, wbd, brow)


def _gather_sum(idxc, table, oc, act, npad, qout):
    nch = idxc.shape[1]
    kk = idxc.shape[2]
    mesh = plsc.VectorSubcoreMesh(core_axis_name="c", subcore_axis_name="s")

    @functools.partial(
        pl.kernel,
        out_type=jax.ShapeDtypeStruct((npad * oc,), jnp.float32),
        mesh=mesh,
        scratch_types=(
            [pltpu.VMEM((kk, BC), jnp.int32)]
            + [pltpu.VMEM((BC, oc), jnp.float32) for _ in range(kk)]
            + [pltpu.VMEM((BC * oc,), jnp.float32),
               pltpu.SemaphoreType.DMA]
        ),
        compiler_params=pltpu.CompilerParams(use_tc_tiling_on_sc=False),
    )
    def run(idx_hbm, table_hbm, out_hbm, idx_v, *rest):
        bufs = rest[:kk]
        out_v = rest[kk]
        sem = rest[kk + 1]
        wid = lax.axis_index("s") * 2 + lax.axis_index("c")
        base0 = wid * (nch * BC)
        for c in range(nch):
            base = base0 + c * BC
            pltpu.sync_copy(idx_hbm.at[wid, c], idx_v)
            cps = [
                pltpu.async_copy(table_hbm.at[idx_v.at[j]], bufs[j], sem)
                for j in range(kk)
            ]
            for cp in cps:
                cp.wait()

            def row(r, carry):
                for c2 in range(oc // 16):
                    s = bufs[0][r, pl.ds(c2 * 16, 16)]
                    for j in range(1, kk):
                        s = s + bufs[j][r, pl.ds(c2 * 16, 16)]
                    if act:
                        s = 1.0 / (1.0 + jnp.exp(-s))
                    out_v[pl.ds(r * oc + c2 * 16, 16)] = s
                return carry

            lax.fori_loop(0, BC, row, 0)
            pltpu.sync_copy(out_v, out_hbm.at[pl.ds(base * oc, BC * oc)])

    return run(idxc, table)


def kernel(features, knn_indices, W0, b0, W1, b1, W2, b2):
    n, _ = features.shape
    k = knn_indices.shape[1]
    nch = -(-n // (NW * BC))
    npad = NW * BC * nch

    idx = jnp.pad(knn_indices, ((0, npad - n), (0, 0)))

    w2p = jnp.pad(W2, ((0, 16 - W2.shape[0]), (0, 0)))
    b2p = jnp.pad(b2, ((0, 0), (0, 16 - b2.shape[1])))

    kr = jnp.arange(k, dtype=jnp.int32)
    eye = {q: jnp.eye(q, dtype=jnp.float32) for q in (1, 2, 4)}
    h = features
    layers = (
        (W0, b0, True, 1, 2),
        (W1, b1, True, 2, 4),
        (w2p, b2p, False, 4, 1),
    )
    for wgt, bias, act, qin, qout in layers:
        oc = wgt.shape[0]
        cin = wgt.shape[1] // k
        g = 128 // oc
        wk = wgt.reshape(oc, k, cin).transpose(1, 2, 0)
        wgrp = wk.reshape(k // g, g, cin, oc).transpose(0, 2, 1, 3)
        wgrp = wgrp.reshape(k // g, cin, 128)
        wbd = jax.vmap(lambda w: jnp.kron(eye[qin], w))(wgrp)
        brow = jnp.tile(jnp.pad(bias, ((0, 0), (0, 128 - oc))), (1, qin))
        idxa = idx * g + ((kr // g) * npad * g + kr % g)[None, :]
        idxc = idxa.reshape(NW, nch, BC, k).transpose(0, 1, 3, 2)
        y = _matmul_tables(h, wbd, brow, npad // qin)
        table = y.reshape(npad * k, oc)
        hflat = _gather_sum(idxc, table, oc, act, npad, qout)
        h = hflat.reshape(npad // qout, qout * oc)

    return h.reshape(npad, 16)[:n, :3]
